# baseline (speedup 1.0000x reference)
import functools

import jax
import jax.numpy as jnp
from jax import lax
from jax.experimental import pallas as pl
from jax.experimental.pallas import tpu as pltpu
from jax.experimental.pallas import tpu_sc as plsc

HIDDEN = 256
HALF = 128
QCOL = 64
NUM_GRAPHS = 64
BN_EPS = 1e-5
N_NODES = 10000
N_EDGES = 160000

E_PAD = 163840
EDGES_PER_TILE = 10240
CHUNKS_PER_TILE = 80
CHUNK = 128
ACC_ROWS = 10240
ROWS_PER_TILE = ACC_ROWS // 16

_NODE_BM = 1000
_EDGE_BM = 2048


def _make_msg_body(qpair):
    def _msg_body(xtab, est, src3, dst3, out, src_v, dst_v, gbuf, ebuf, acc,
                  sem):
        c = lax.axis_index("c")
        s = lax.axis_index("s")
        q = c + 2 * qpair

        pltpu.sync_copy(src3.at[s], src_v)
        pltpu.sync_copy(dst3.at[s], dst_v)

        off = q * N_NODES

        def adj(i, carry):
            for j in range(8):
                sl = pl.ds(j * 16, 16)
                src_v[i, sl] = src_v[i, sl] + off
            return carry

        lax.fori_loop(0, CHUNKS_PER_TILE, adj, 0)

        def zrow(i, carry):
            for j in range(QCOL // 16):
                gbuf[i, pl.ds(j * 16, 16)] = jnp.zeros((16,), jnp.float32)
            return carry

        lax.fori_loop(0, CHUNK, zrow, 0)
        for w in range(ROWS_PER_TILE // CHUNK):
            pltpu.sync_copy(gbuf, acc.at[pl.ds(s * ROWS_PER_TILE + w * CHUNK,
                                               CHUNK)])
        plsc.subcore_barrier()

        ebase = q * E_PAD + s * EDGES_PER_TILE

        def chunk(k, carry):
            pltpu.async_copy(xtab.at[src_v.at[k]], gbuf, sem).wait()
            pltpu.sync_copy(est.at[pl.ds(ebase + k * CHUNK, CHUNK)], ebuf)

            def row(i, carry2):
                for j in range(QCOL // 16):
                    sl = pl.ds(j * 16, 16)
                    gbuf[i, sl] = jnp.maximum(gbuf[i, sl] + ebuf[i, sl], 0.0)
                return carry2

            lax.fori_loop(0, CHUNK, row, 0)
            pltpu.sync_copy(gbuf, acc.at[dst_v.at[k]], add=True)
            return carry

        lax.fori_loop(0, CHUNKS_PER_TILE, chunk, 0)
        plsc.subcore_barrier()

        rbase = s * ROWS_PER_TILE
        pltpu.sync_copy(acc.at[pl.ds(rbase, ROWS_PER_TILE)],
                        out.at[c, pl.ds(rbase, ROWS_PER_TILE)])

    return _msg_body


def _message_pass(xtab, est, src3, dst3, qpair):
    k = functools.partial(
        pl.kernel,
        mesh=plsc.VectorSubcoreMesh(core_axis_name="c", subcore_axis_name="s"),
        compiler_params=pltpu.CompilerParams(use_tc_tiling_on_sc=False),
        out_type=jax.ShapeDtypeStruct((2, ACC_ROWS, QCOL), jnp.float32),
        scratch_types=[
            pltpu.VMEM((CHUNKS_PER_TILE, CHUNK), jnp.int32),
            pltpu.VMEM((CHUNKS_PER_TILE, CHUNK), jnp.int32),
            pltpu.VMEM((CHUNK, QCOL), jnp.float32),
            pltpu.VMEM((CHUNK, QCOL), jnp.float32),
            pltpu.VMEM_SHARED((ACC_ROWS, QCOL), jnp.float32),
            pltpu.SemaphoreType.DMA,
        ],
    )(_make_msg_body(qpair))
    return k(xtab, est, src3, dst3)


def _nproj_body(x_ref, w_ref, b_ref, lo_ref, hi_ref):
    x = x_ref[...]
    x = jnp.where(x == x, x, 0.0)
    o = jax.nn.relu(
        jnp.dot(x, w_ref[...], preferred_element_type=jnp.float32) + b_ref[...]
    )
    lo_ref[...] = o[:, :HALF]
    hi_ref[...] = o[:, HALF:]


def _node_proj(x, w, b):
    rows, k = x.shape
    grid = rows // _NODE_BM
    return pl.pallas_call(
        _nproj_body,
        grid=(grid,),
        in_specs=[
            pl.BlockSpec((_NODE_BM, k), lambda i: (i, 0)),
            pl.BlockSpec((k, HIDDEN), lambda i: (0, 0)),
            pl.BlockSpec((1, HIDDEN), lambda i: (0, 0)),
        ],
        out_specs=[
            pl.BlockSpec((_NODE_BM, HALF), lambda i: (i, 0)),
            pl.BlockSpec((_NODE_BM, HALF), lambda i: (i, 0)),
        ],
        out_shape=[
            jax.ShapeDtypeStruct((rows, HALF), jnp.float32),
            jax.ShapeDtypeStruct((rows, HALF), jnp.float32),
        ],
    )(x, w, b)


def _eproj_body(a_ref, w_ref, b_ref, o_ref):
    a = a_ref[...]
    a = jnp.where(a == a, a, 0.0)
    o_ref[...] = jax.nn.relu(
        jnp.dot(a, w_ref[0], preferred_element_type=jnp.float32) + b_ref[0]
    )


def _edge_proj(e_attr_pad, wq, bq):
    k = e_attr_pad.shape[1]
    nblk = E_PAD // _EDGE_BM
    return pl.pallas_call(
        _eproj_body,
        grid=(4, nblk),
        in_specs=[
            pl.BlockSpec((_EDGE_BM, k), lambda q, i: (i, 0)),
            pl.BlockSpec((1, k, QCOL), lambda q, i: (q, 0, 0)),
            pl.BlockSpec((1, 1, QCOL), lambda q, i: (q, 0, 0)),
        ],
        out_specs=pl.BlockSpec((_EDGE_BM, QCOL), lambda q, i: (q * nblk + i, 0)),
        out_shape=jax.ShapeDtypeStruct((4 * E_PAD, QCOL), jnp.float32),
    )(e_attr_pad, wq, bq)


def _mlp_body(xlo_ref, xhi_ref, a0_ref, a1_ref, a2_ref, a3_ref, w1_ref,
              b1_ref, w2_ref, b2_ref, olo_ref, ohi_ref):
    aggr = jnp.concatenate(
        [a0_ref[0], a1_ref[0], a2_ref[0], a3_ref[0]], axis=1)
    h = jnp.concatenate([xlo_ref[...], xhi_ref[...]], axis=1) + aggr
    h = jax.nn.relu(
        jnp.dot(h, w1_ref[...], preferred_element_type=jnp.float32) + b1_ref[...]
    )
    o = jnp.dot(h, w2_ref[...], preferred_element_type=jnp.float32) + b2_ref[...]
    o = jax.nn.relu(o)
    olo_ref[...] = o[:, :HALF]
    ohi_ref[...] = o[:, HALF:]


def _mlp(xlo, xhi, aggr01, aggr23, w1t, b1, w2t, b2):
    grid = N_NODES // _NODE_BM
    h1 = w1t.shape[1]
    half_spec = pl.BlockSpec((_NODE_BM, HALF), lambda i: (i, 0))

    def quarter(c):
        return pl.BlockSpec((1, _NODE_BM, QCOL), lambda i, c=c: (c, i, 0))

    return pl.pallas_call(
        _mlp_body,
        grid=(grid,),
        in_specs=[
            half_spec,
            half_spec,
            quarter(0),
            quarter(1),
            quarter(0),
            quarter(1),
            pl.BlockSpec((HIDDEN, h1), lambda i: (0, 0)),
            pl.BlockSpec((1, h1), lambda i: (0, 0)),
            pl.BlockSpec((h1, HIDDEN), lambda i: (0, 0)),
            pl.BlockSpec((1, HIDDEN), lambda i: (0, 0)),
        ],
        out_specs=[half_spec, half_spec],
        out_shape=[
            jax.ShapeDtypeStruct((N_NODES, HALF), jnp.float32),
            jax.ShapeDtypeStruct((N_NODES, HALF), jnp.float32),
        ],
    )(xlo, xhi, aggr01, aggr01, aggr23, aggr23, w1t, b1, w2t, b2)


def _pool_body(xlo_ref, xhi_ref, seg_ref, o_ref, sum_s, cnt_s):
    i = pl.program_id(0)

    @pl.when(i == 0)
    def _():
        sum_s[...] = jnp.zeros_like(sum_s)
        cnt_s[...] = jnp.zeros_like(cnt_s)

    seg = seg_ref[0, 0, :]
    bm = seg.shape[0]
    onehot = (
        seg[None, :]
        == jax.lax.broadcasted_iota(jnp.int32, (NUM_GRAPHS, bm), 0)
    ).astype(jnp.float32)
    x = jnp.concatenate([xlo_ref[...], xhi_ref[...]], axis=1)
    sum_s[...] += jnp.dot(onehot, x, preferred_element_type=jnp.float32)
    cnt_s[...] += jnp.dot(
        onehot, jnp.ones((bm, HIDDEN), jnp.float32),
        preferred_element_type=jnp.float32,
    )

    @pl.when(i == pl.num_programs(0) - 1)
    def _():
        o_ref[...] = sum_s[...] / jnp.maximum(cnt_s[...], 1.0)


def _pool(xlo, xhi, batch_ids):
    grid = N_NODES // _NODE_BM
    seg3 = batch_ids.reshape(grid, 1, _NODE_BM)
    half_spec = pl.BlockSpec((_NODE_BM, HALF), lambda i: (i, 0))
    return pl.pallas_call(
        _pool_body,
        grid=(grid,),
        in_specs=[
            half_spec,
            half_spec,
            pl.BlockSpec((1, 1, _NODE_BM), lambda i: (i, 0, 0)),
        ],
        out_specs=pl.BlockSpec((NUM_GRAPHS, HIDDEN), lambda i: (0, 0)),
        out_shape=jax.ShapeDtypeStruct((NUM_GRAPHS, HIDDEN), jnp.float32),
        scratch_shapes=[
            pltpu.VMEM((NUM_GRAPHS, HIDDEN), jnp.float32),
            pltpu.VMEM((NUM_GRAPHS, HIDDEN), jnp.float32),
        ],
    )(xlo, xhi, seg3)


def _encoder(params, x_in, edge_index, edge_attr, batch_ids):
    npw_t = params['node_proj_w'].T
    epw_t = params['edge_proj_w'].T
    ek = edge_attr.shape[1]
    e_attr_pad = jnp.concatenate(
        [edge_attr, jnp.zeros((E_PAD - N_EDGES, ek), jnp.float32)], axis=0)
    src3 = jnp.concatenate(
        [edge_index[0], jnp.zeros((E_PAD - N_EDGES,), jnp.int32)]
    ).reshape(16, CHUNKS_PER_TILE, CHUNK)
    dst3 = jnp.concatenate(
        [edge_index[1],
         jnp.full((E_PAD - N_EDGES,), N_NODES, jnp.int32)]
    ).reshape(16, CHUNKS_PER_TILE, CHUNK)

    xlo, xhi = _node_proj(x_in, npw_t, params['node_proj_b'][None, :])
    epw_q = jnp.stack(
        [epw_t[:, i * QCOL:(i + 1) * QCOL] for i in range(4)])
    epb_q = jnp.stack(
        [params['edge_proj_b'][None, i * QCOL:(i + 1) * QCOL]
         for i in range(4)])
    est = _edge_proj(e_attr_pad, epw_q, epb_q)

    for layer in params['layers']:
        scale = layer['bn_gamma'] * jax.lax.rsqrt(layer['bn_var'] + BN_EPS)
        shift = layer['bn_beta'] - layer['bn_mean'] * scale
        w1t = layer['w1'].T
        w2t = layer['w2'].T * scale[None, :]
        b2 = (layer['b2'] * scale + shift)[None, :]
        xtab = jnp.concatenate(
            [xlo[:, :QCOL], xlo[:, QCOL:], xhi[:, :QCOL], xhi[:, QCOL:]],
            axis=0)
        aggr01 = _message_pass(xtab, est, src3, dst3, 0)
        aggr23 = _message_pass(xtab, est, src3, dst3, 1)
        xlo, xhi = _mlp(xlo, xhi, aggr01, aggr23, w1t,
                        layer['b1'][None, :], w2t, b2)
    return _pool(xlo, xhi, batch_ids)


def kernel(prot_x, prot_edge_index, prot_edge_attr, prot_batch,
           drug_x, drug_edge_index, drug_edge_attr, drug_batch,
           prot_params, drug_params):
    p = _encoder(prot_params, prot_x, prot_edge_index, prot_edge_attr, prot_batch)
    d = _encoder(drug_params, drug_x, drug_edge_index, drug_edge_attr, drug_batch)
    return (p, d)


# speedup vs baseline: 1.4192x; 1.4192x over previous
import functools

import jax
import jax.numpy as jnp
from jax import lax
from jax.experimental import pallas as pl
from jax.experimental.pallas import tpu as pltpu
from jax.experimental.pallas import tpu_sc as plsc

HIDDEN = 256
HALF = 128
QCOL = 64
NUM_GRAPHS = 64
BN_EPS = 1e-5
N_NODES = 10000
N_EDGES = 160000

E_PAD = 163840
EDGES_PER_TILE = 10240
CHUNKS_PER_TILE = 80
CHUNK = 128
ACC_ROWS = 10240
ROWS_PER_TILE = ACC_ROWS // 16

_NODE_BM = 1000
_EDGE_BM = 2048


def _msg_body(xtab, est, src3, dst3, out, src_v, dst_v, g0, g1, e0, e1, acc,
              sg0, sg1, se0, se1):
    c = lax.axis_index("c")
    s = lax.axis_index("s")
    gb = (g0, g1)
    eb = (e0, e1)
    sg = (sg0, sg1)
    se = (se0, se1)

    pltpu.sync_copy(src3.at[s], src_v)
    pltpu.sync_copy(dst3.at[s], dst_v)

    def adj(off):
        def body(i, carry):
            for j in range(8):
                sl = pl.ds(j * 16, 16)
                src_v[i, sl] = src_v[i, sl] + off
            return carry
        lax.fori_loop(0, CHUNKS_PER_TILE, body, 0)

    adj(c * N_NODES)

    rbase = s * ROWS_PER_TILE

    for qp in range(2):
        if qp:
            adj(2 * N_NODES)

        def zrow(i, carry):
            for j in range(QCOL // 16):
                g0[i, pl.ds(j * 16, 16)] = jnp.zeros((16,), jnp.float32)
            return carry

        lax.fori_loop(0, CHUNK, zrow, 0)
        for w in range(ROWS_PER_TILE // CHUNK):
            pltpu.sync_copy(g0, acc.at[pl.ds(rbase + w * CHUNK, CHUNK)])
        plsc.subcore_barrier()

        ebase = (c + 2 * qp) * E_PAD + s * EDGES_PER_TILE

        def start(kk, b):
            pltpu.async_copy(xtab.at[src_v.at[kk]], gb[b], sg[b])
            pltpu.async_copy(est.at[pl.ds(ebase + kk * CHUNK, CHUNK)],
                             eb[b], se[b])

        def drain(kk, b):
            pltpu.make_async_copy(xtab.at[src_v.at[kk]], gb[b], sg[b]).wait()
            pltpu.make_async_copy(est.at[pl.ds(ebase + kk * CHUNK, CHUNK)],
                                  eb[b], se[b]).wait()

        start(0, 0)
        start(1, 1)

        def pair(k, carry):
            for b in range(2):
                kk = 2 * k + b
                drain(kk, b)

                def row(i, carry2):
                    for r in range(2):
                        for j in range(QCOL // 16):
                            sl = pl.ds(j * 16, 16)
                            gb[b][2 * i + r, sl] = jnp.maximum(
                                gb[b][2 * i + r, sl] + eb[b][2 * i + r, sl],
                                0.0)
                    return carry2

                lax.fori_loop(0, CHUNK // 2, row, 0)
                pltpu.sync_copy(gb[b], acc.at[dst_v.at[kk]], add=True)

                @pl.when(kk + 2 < CHUNKS_PER_TILE)
                def _():
                    start(kk + 2, b)
            return carry

        lax.fori_loop(0, CHUNKS_PER_TILE // 2, pair, 0)
        plsc.subcore_barrier()

        q = c + 2 * qp
        pltpu.sync_copy(acc.at[pl.ds(rbase, ROWS_PER_TILE)],
                        out.at[q, pl.ds(rbase, ROWS_PER_TILE)])
        plsc.subcore_barrier()


def _message_pass(xtab, est, src3, dst3):
    k = functools.partial(
        pl.kernel,
        mesh=plsc.VectorSubcoreMesh(core_axis_name="c", subcore_axis_name="s"),
        compiler_params=pltpu.CompilerParams(use_tc_tiling_on_sc=False),
        out_type=jax.ShapeDtypeStruct((4, ACC_ROWS, QCOL), jnp.float32),
        scratch_types=[
            pltpu.VMEM((CHUNKS_PER_TILE, CHUNK), jnp.int32),
            pltpu.VMEM((CHUNKS_PER_TILE, CHUNK), jnp.int32),
            pltpu.VMEM((CHUNK, QCOL), jnp.float32),
            pltpu.VMEM((CHUNK, QCOL), jnp.float32),
            pltpu.VMEM((CHUNK, QCOL), jnp.float32),
            pltpu.VMEM((CHUNK, QCOL), jnp.float32),
            pltpu.VMEM_SHARED((ACC_ROWS, QCOL), jnp.float32),
            pltpu.SemaphoreType.DMA,
            pltpu.SemaphoreType.DMA,
            pltpu.SemaphoreType.DMA,
            pltpu.SemaphoreType.DMA,
        ],
    )(_msg_body)
    return k(xtab, est, src3, dst3)


def _nproj_body(x_ref, w_ref, b_ref, lo_ref, hi_ref):
    x = x_ref[...]
    x = jnp.where(x == x, x, 0.0)
    o = jax.nn.relu(
        jnp.dot(x, w_ref[...], preferred_element_type=jnp.float32) + b_ref[...]
    )
    lo_ref[...] = o[:, :HALF]
    hi_ref[...] = o[:, HALF:]


def _node_proj(x, w, b):
    rows, k = x.shape
    grid = rows // _NODE_BM
    return pl.pallas_call(
        _nproj_body,
        grid=(grid,),
        in_specs=[
            pl.BlockSpec((_NODE_BM, k), lambda i: (i, 0)),
            pl.BlockSpec((k, HIDDEN), lambda i: (0, 0)),
            pl.BlockSpec((1, HIDDEN), lambda i: (0, 0)),
        ],
        out_specs=[
            pl.BlockSpec((_NODE_BM, HALF), lambda i: (i, 0)),
            pl.BlockSpec((_NODE_BM, HALF), lambda i: (i, 0)),
        ],
        out_shape=[
            jax.ShapeDtypeStruct((rows, HALF), jnp.float32),
            jax.ShapeDtypeStruct((rows, HALF), jnp.float32),
        ],
    )(x, w, b)


def _eproj_body(a_ref, w_ref, b_ref, o_ref):
    a = a_ref[...]
    a = jnp.where(a == a, a, 0.0)
    o_ref[...] = jax.nn.relu(
        jnp.dot(a, w_ref[0], preferred_element_type=jnp.float32) + b_ref[0]
    )


def _edge_proj(e_attr_pad, wq, bq):
    k = e_attr_pad.shape[1]
    nblk = E_PAD // _EDGE_BM
    return pl.pallas_call(
        _eproj_body,
        grid=(4, nblk),
        in_specs=[
            pl.BlockSpec((_EDGE_BM, k), lambda q, i: (i, 0)),
            pl.BlockSpec((1, k, QCOL), lambda q, i: (q, 0, 0)),
            pl.BlockSpec((1, 1, QCOL), lambda q, i: (q, 0, 0)),
        ],
        out_specs=pl.BlockSpec((_EDGE_BM, QCOL), lambda q, i: (q * nblk + i, 0)),
        out_shape=jax.ShapeDtypeStruct((4 * E_PAD, QCOL), jnp.float32),
    )(e_attr_pad, wq, bq)


def _mlp_body(xlo_ref, xhi_ref, a0_ref, a1_ref, a2_ref, a3_ref, w1_ref,
              b1_ref, w2_ref, b2_ref, olo_ref, ohi_ref):
    aggr = jnp.concatenate(
        [a0_ref[0], a1_ref[0], a2_ref[0], a3_ref[0]], axis=1)
    h = jnp.concatenate([xlo_ref[...], xhi_ref[...]], axis=1) + aggr
    h = jax.nn.relu(
        jnp.dot(h, w1_ref[...], preferred_element_type=jnp.float32) + b1_ref[...]
    )
    o = jnp.dot(h, w2_ref[...], preferred_element_type=jnp.float32) + b2_ref[...]
    o = jax.nn.relu(o)
    olo_ref[...] = o[:, :HALF]
    ohi_ref[...] = o[:, HALF:]


def _mlp(xlo, xhi, aggr4, w1t, b1, w2t, b2):
    grid = N_NODES // _NODE_BM
    h1 = w1t.shape[1]
    half_spec = pl.BlockSpec((_NODE_BM, HALF), lambda i: (i, 0))

    def quarter(q):
        return pl.BlockSpec((1, _NODE_BM, QCOL), lambda i, q=q: (q, i, 0))

    return pl.pallas_call(
        _mlp_body,
        grid=(grid,),
        in_specs=[
            half_spec,
            half_spec,
            quarter(0),
            quarter(1),
            quarter(2),
            quarter(3),
            pl.BlockSpec((HIDDEN, h1), lambda i: (0, 0)),
            pl.BlockSpec((1, h1), lambda i: (0, 0)),
            pl.BlockSpec((h1, HIDDEN), lambda i: (0, 0)),
            pl.BlockSpec((1, HIDDEN), lambda i: (0, 0)),
        ],
        out_specs=[half_spec, half_spec],
        out_shape=[
            jax.ShapeDtypeStruct((N_NODES, HALF), jnp.float32),
            jax.ShapeDtypeStruct((N_NODES, HALF), jnp.float32),
        ],
    )(xlo, xhi, aggr4, aggr4, aggr4, aggr4, w1t, b1, w2t, b2)


def _pool_body(xlo_ref, xhi_ref, seg_ref, o_ref, sum_s, cnt_s):
    i = pl.program_id(0)

    @pl.when(i == 0)
    def _():
        sum_s[...] = jnp.zeros_like(sum_s)
        cnt_s[...] = jnp.zeros_like(cnt_s)

    seg = seg_ref[0, 0, :]
    bm = seg.shape[0]
    onehot = (
        seg[None, :]
        == jax.lax.broadcasted_iota(jnp.int32, (NUM_GRAPHS, bm), 0)
    ).astype(jnp.float32)
    x = jnp.concatenate([xlo_ref[...], xhi_ref[...]], axis=1)
    sum_s[...] += jnp.dot(onehot, x, preferred_element_type=jnp.float32)
    cnt_s[...] += jnp.dot(
        onehot, jnp.ones((bm, HIDDEN), jnp.float32),
        preferred_element_type=jnp.float32,
    )

    @pl.when(i == pl.num_programs(0) - 1)
    def _():
        o_ref[...] = sum_s[...] / jnp.maximum(cnt_s[...], 1.0)


def _pool(xlo, xhi, batch_ids):
    grid = N_NODES // _NODE_BM
    seg3 = batch_ids.reshape(grid, 1, _NODE_BM)
    half_spec = pl.BlockSpec((_NODE_BM, HALF), lambda i: (i, 0))
    return pl.pallas_call(
        _pool_body,
        grid=(grid,),
        in_specs=[
            half_spec,
            half_spec,
            pl.BlockSpec((1, 1, _NODE_BM), lambda i: (i, 0, 0)),
        ],
        out_specs=pl.BlockSpec((NUM_GRAPHS, HIDDEN), lambda i: (0, 0)),
        out_shape=jax.ShapeDtypeStruct((NUM_GRAPHS, HIDDEN), jnp.float32),
        scratch_shapes=[
            pltpu.VMEM((NUM_GRAPHS, HIDDEN), jnp.float32),
            pltpu.VMEM((NUM_GRAPHS, HIDDEN), jnp.float32),
        ],
    )(xlo, xhi, seg3)


def _encoder_prep(params, x_in, edge_index, edge_attr):
    npw_t = params['node_proj_w'].T
    epw_t = params['edge_proj_w'].T
    ek = edge_attr.shape[1]
    e_attr_pad = jnp.concatenate(
        [edge_attr, jnp.zeros((E_PAD - N_EDGES, ek), jnp.float32)], axis=0)
    src3 = jnp.concatenate(
        [edge_index[0], jnp.zeros((E_PAD - N_EDGES,), jnp.int32)]
    ).reshape(16, CHUNKS_PER_TILE, CHUNK)
    dst3 = jnp.concatenate(
        [edge_index[1],
         jnp.full((E_PAD - N_EDGES,), N_NODES, jnp.int32)]
    ).reshape(16, CHUNKS_PER_TILE, CHUNK)

    xlo, xhi = _node_proj(x_in, npw_t, params['node_proj_b'][None, :])
    epw_q = jnp.stack(
        [epw_t[:, i * QCOL:(i + 1) * QCOL] for i in range(4)])
    epb_q = jnp.stack(
        [params['edge_proj_b'][None, i * QCOL:(i + 1) * QCOL]
         for i in range(4)])
    est = _edge_proj(e_attr_pad, epw_q, epb_q)
    return {'xlo': xlo, 'xhi': xhi, 'est': est, 'src3': src3, 'dst3': dst3}


def _layer_weights(layer):
    scale = layer['bn_gamma'] * jax.lax.rsqrt(layer['bn_var'] + BN_EPS)
    shift = layer['bn_beta'] - layer['bn_mean'] * scale
    w1t = layer['w1'].T
    w2t = layer['w2'].T * scale[None, :]
    b2 = (layer['b2'] * scale + shift)[None, :]
    return w1t, layer['b1'][None, :], w2t, b2


def _layer_msg(st):
    xtab = jnp.concatenate(
        [st['xlo'][:, :QCOL], st['xlo'][:, QCOL:],
         st['xhi'][:, :QCOL], st['xhi'][:, QCOL:]], axis=0)
    return _message_pass(xtab, st['est'], st['src3'], st['dst3'])


def kernel(prot_x, prot_edge_index, prot_edge_attr, prot_batch,
           drug_x, drug_edge_index, drug_edge_attr, drug_batch,
           prot_params, drug_params):
    sp = _encoder_prep(prot_params, prot_x, prot_edge_index, prot_edge_attr)
    sd = _encoder_prep(drug_params, drug_x, drug_edge_index, drug_edge_attr)
    for lp, ld in zip(prot_params['layers'], drug_params['layers']):
        ap = _layer_msg(sp)
        ad = _layer_msg(sd)
        sp['xlo'], sp['xhi'] = _mlp(sp['xlo'], sp['xhi'], ap,
                                    *_layer_weights(lp))
        sd['xlo'], sd['xhi'] = _mlp(sd['xlo'], sd['xhi'], ad,
                                    *_layer_weights(ld))
    p = _pool(sp['xlo'], sp['xhi'], prot_batch)
    d = _pool(sd['xlo'], sd['xhi'], drug_batch)
    return (p, d)


# speedup vs baseline: 1.4197x; 1.0004x over previous
import functools

import jax
import jax.numpy as jnp
from jax import lax
from jax.experimental import pallas as pl
from jax.experimental.pallas import tpu as pltpu
from jax.experimental.pallas import tpu_sc as plsc

HIDDEN = 256
HALF = 128
QCOL = 64
NUM_GRAPHS = 64
BN_EPS = 1e-5
N_NODES = 10000
N_EDGES = 160000

E_PAD = 163840
EDGES_PER_TILE = 10240
CHUNKS_PER_TILE = 80
CHUNK = 128
ACC_ROWS = 10240
ROWS_PER_TILE = ACC_ROWS // 16

_NODE_BM = 1000
_EDGE_BM = 2048


NBUF = 2


def _msg_body(xtab, est, src3, dst3, out, src_v, dst_v,
              g0, g1, e0, e1, acc, sg0, sg1, se0, se1):
    c = lax.axis_index("c")
    s = lax.axis_index("s")
    gb = (g0, g1)
    eb = (e0, e1)
    sg = (sg0, sg1)
    se = (se0, se1)

    pltpu.sync_copy(src3.at[s], src_v)
    pltpu.sync_copy(dst3.at[s], dst_v)

    def adj(off):
        def body(i, carry):
            for j in range(8):
                sl = pl.ds(j * 16, 16)
                src_v[i, sl] = src_v[i, sl] + off
            return carry
        lax.fori_loop(0, CHUNKS_PER_TILE, body, 0)

    adj(c * N_NODES)

    rbase = s * ROWS_PER_TILE

    for qp in range(2):
        if qp:
            adj(2 * N_NODES)

        def zrow(i, carry):
            for j in range(QCOL // 16):
                g0[i, pl.ds(j * 16, 16)] = jnp.zeros((16,), jnp.float32)
            return carry

        lax.fori_loop(0, CHUNK, zrow, 0)
        for w in range(ROWS_PER_TILE // CHUNK):
            pltpu.sync_copy(g0, acc.at[pl.ds(rbase + w * CHUNK, CHUNK)])
        plsc.subcore_barrier()

        ebase = (c + 2 * qp) * E_PAD + s * EDGES_PER_TILE

        def start(kk, b):
            pltpu.async_copy(xtab.at[src_v.at[kk]], gb[b], sg[b])
            pltpu.async_copy(est.at[pl.ds(ebase + kk * CHUNK, CHUNK)],
                             eb[b], se[b])

        def drain(kk, b):
            pltpu.make_async_copy(xtab.at[src_v.at[kk]], gb[b], sg[b]).wait()
            pltpu.make_async_copy(est.at[pl.ds(ebase + kk * CHUNK, CHUNK)],
                                  eb[b], se[b]).wait()

        start(0, 0)
        start(1, 1)

        def quad(k, carry):
            for b in range(NBUF):
                kk = NBUF * k + b
                drain(kk, b)

                def row(i, carry2):
                    for r in range(4):
                        for j in range(QCOL // 16):
                            sl = pl.ds(j * 16, 16)
                            gb[b][4 * i + r, sl] = jnp.maximum(
                                gb[b][4 * i + r, sl] + eb[b][4 * i + r, sl],
                                0.0)
                    return carry2

                lax.fori_loop(0, CHUNK // 4, row, 0)
                pltpu.sync_copy(gb[b], acc.at[dst_v.at[kk]], add=True)

                @pl.when(kk + NBUF < CHUNKS_PER_TILE)
                def _():
                    start(kk + NBUF, b)
            return carry

        lax.fori_loop(0, CHUNKS_PER_TILE // NBUF, quad, 0)
        plsc.subcore_barrier()

        q = c + 2 * qp
        pltpu.sync_copy(acc.at[pl.ds(rbase, ROWS_PER_TILE)],
                        out.at[q, pl.ds(rbase, ROWS_PER_TILE)])
        plsc.subcore_barrier()


def _message_pass(xtab, est, src3, dst3):
    k = functools.partial(
        pl.kernel,
        mesh=plsc.VectorSubcoreMesh(core_axis_name="c", subcore_axis_name="s"),
        compiler_params=pltpu.CompilerParams(use_tc_tiling_on_sc=False),
        out_type=jax.ShapeDtypeStruct((4, ACC_ROWS, QCOL), jnp.float32),
        scratch_types=(
            [pltpu.VMEM((CHUNKS_PER_TILE, CHUNK), jnp.int32)] * 2
            + [pltpu.VMEM((CHUNK, QCOL), jnp.float32)] * (2 * NBUF)
            + [pltpu.VMEM_SHARED((ACC_ROWS, QCOL), jnp.float32)]
            + [pltpu.SemaphoreType.DMA] * (2 * NBUF)
        ),
    )(_msg_body)
    return k(xtab, est, src3, dst3)


def _nproj_body(x_ref, w_ref, b_ref, lo_ref, hi_ref):
    x = x_ref[...]
    x = jnp.where(x == x, x, 0.0)
    o = jax.nn.relu(
        jnp.dot(x, w_ref[...], preferred_element_type=jnp.float32) + b_ref[...]
    )
    lo_ref[...] = o[:, :HALF]
    hi_ref[...] = o[:, HALF:]


def _node_proj(x, w, b):
    rows, k = x.shape
    grid = rows // _NODE_BM
    return pl.pallas_call(
        _nproj_body,
        grid=(grid,),
        in_specs=[
            pl.BlockSpec((_NODE_BM, k), lambda i: (i, 0)),
            pl.BlockSpec((k, HIDDEN), lambda i: (0, 0)),
            pl.BlockSpec((1, HIDDEN), lambda i: (0, 0)),
        ],
        out_specs=[
            pl.BlockSpec((_NODE_BM, HALF), lambda i: (i, 0)),
            pl.BlockSpec((_NODE_BM, HALF), lambda i: (i, 0)),
        ],
        out_shape=[
            jax.ShapeDtypeStruct((rows, HALF), jnp.float32),
            jax.ShapeDtypeStruct((rows, HALF), jnp.float32),
        ],
    )(x, w, b)


def _eproj_body(a_ref, w_ref, b_ref, o_ref):
    a = a_ref[...]
    a = jnp.where(a == a, a, 0.0)
    o_ref[...] = jax.nn.relu(
        jnp.dot(a, w_ref[0], preferred_element_type=jnp.float32) + b_ref[0]
    )


def _edge_proj(e_attr_pad, wq, bq):
    k = e_attr_pad.shape[1]
    nblk = E_PAD // _EDGE_BM
    return pl.pallas_call(
        _eproj_body,
        grid=(4, nblk),
        in_specs=[
            pl.BlockSpec((_EDGE_BM, k), lambda q, i: (i, 0)),
            pl.BlockSpec((1, k, QCOL), lambda q, i: (q, 0, 0)),
            pl.BlockSpec((1, 1, QCOL), lambda q, i: (q, 0, 0)),
        ],
        out_specs=pl.BlockSpec((_EDGE_BM, QCOL), lambda q, i: (q * nblk + i, 0)),
        out_shape=jax.ShapeDtypeStruct((4 * E_PAD, QCOL), jnp.float32),
    )(e_attr_pad, wq, bq)


def _mlp_body(xlo_ref, xhi_ref, a0_ref, a1_ref, a2_ref, a3_ref, w1_ref,
              b1_ref, w2_ref, b2_ref, olo_ref, ohi_ref):
    aggr = jnp.concatenate(
        [a0_ref[0], a1_ref[0], a2_ref[0], a3_ref[0]], axis=1)
    h = jnp.concatenate([xlo_ref[...], xhi_ref[...]], axis=1) + aggr
    h = jax.nn.relu(
        jnp.dot(h, w1_ref[...], preferred_element_type=jnp.float32) + b1_ref[...]
    )
    o = jnp.dot(h, w2_ref[...], preferred_element_type=jnp.float32) + b2_ref[...]
    o = jax.nn.relu(o)
    olo_ref[...] = o[:, :HALF]
    ohi_ref[...] = o[:, HALF:]


def _mlp(xlo, xhi, aggr4, w1t, b1, w2t, b2):
    grid = N_NODES // _NODE_BM
    h1 = w1t.shape[1]
    half_spec = pl.BlockSpec((_NODE_BM, HALF), lambda i: (i, 0))

    def quarter(q):
        return pl.BlockSpec((1, _NODE_BM, QCOL), lambda i, q=q: (q, i, 0))

    return pl.pallas_call(
        _mlp_body,
        grid=(grid,),
        in_specs=[
            half_spec,
            half_spec,
            quarter(0),
            quarter(1),
            quarter(2),
            quarter(3),
            pl.BlockSpec((HIDDEN, h1), lambda i: (0, 0)),
            pl.BlockSpec((1, h1), lambda i: (0, 0)),
            pl.BlockSpec((h1, HIDDEN), lambda i: (0, 0)),
            pl.BlockSpec((1, HIDDEN), lambda i: (0, 0)),
        ],
        out_specs=[half_spec, half_spec],
        out_shape=[
            jax.ShapeDtypeStruct((N_NODES, HALF), jnp.float32),
            jax.ShapeDtypeStruct((N_NODES, HALF), jnp.float32),
        ],
    )(xlo, xhi, aggr4, aggr4, aggr4, aggr4, w1t, b1, w2t, b2)


def _pool_body(xlo_ref, xhi_ref, seg_ref, o_ref, sum_s, cnt_s):
    i = pl.program_id(0)

    @pl.when(i == 0)
    def _():
        sum_s[...] = jnp.zeros_like(sum_s)
        cnt_s[...] = jnp.zeros_like(cnt_s)

    seg = seg_ref[0, 0, :]
    bm = seg.shape[0]
    onehot = (
        seg[None, :]
        == jax.lax.broadcasted_iota(jnp.int32, (NUM_GRAPHS, bm), 0)
    ).astype(jnp.float32)
    x = jnp.concatenate([xlo_ref[...], xhi_ref[...]], axis=1)
    sum_s[...] += jnp.dot(onehot, x, preferred_element_type=jnp.float32)
    cnt_s[...] += jnp.dot(
        onehot, jnp.ones((bm, HIDDEN), jnp.float32),
        preferred_element_type=jnp.float32,
    )

    @pl.when(i == pl.num_programs(0) - 1)
    def _():
        o_ref[...] = sum_s[...] / jnp.maximum(cnt_s[...], 1.0)


def _pool(xlo, xhi, batch_ids):
    grid = N_NODES // _NODE_BM
    seg3 = batch_ids.reshape(grid, 1, _NODE_BM)
    half_spec = pl.BlockSpec((_NODE_BM, HALF), lambda i: (i, 0))
    return pl.pallas_call(
        _pool_body,
        grid=(grid,),
        in_specs=[
            half_spec,
            half_spec,
            pl.BlockSpec((1, 1, _NODE_BM), lambda i: (i, 0, 0)),
        ],
        out_specs=pl.BlockSpec((NUM_GRAPHS, HIDDEN), lambda i: (0, 0)),
        out_shape=jax.ShapeDtypeStruct((NUM_GRAPHS, HIDDEN), jnp.float32),
        scratch_shapes=[
            pltpu.VMEM((NUM_GRAPHS, HIDDEN), jnp.float32),
            pltpu.VMEM((NUM_GRAPHS, HIDDEN), jnp.float32),
        ],
    )(xlo, xhi, seg3)


def _encoder_prep(params, x_in, edge_index, edge_attr):
    npw_t = params['node_proj_w'].T
    epw_t = params['edge_proj_w'].T
    ek = edge_attr.shape[1]
    e_attr_pad = jnp.concatenate(
        [edge_attr, jnp.zeros((E_PAD - N_EDGES, ek), jnp.float32)], axis=0)
    src3 = jnp.concatenate(
        [edge_index[0], jnp.zeros((E_PAD - N_EDGES,), jnp.int32)]
    ).reshape(16, CHUNKS_PER_TILE, CHUNK)
    dst3 = jnp.concatenate(
        [edge_index[1],
         jnp.full((E_PAD - N_EDGES,), N_NODES, jnp.int32)]
    ).reshape(16, CHUNKS_PER_TILE, CHUNK)

    xlo, xhi = _node_proj(x_in, npw_t, params['node_proj_b'][None, :])
    epw_q = jnp.stack(
        [epw_t[:, i * QCOL:(i + 1) * QCOL] for i in range(4)])
    epb_q = jnp.stack(
        [params['edge_proj_b'][None, i * QCOL:(i + 1) * QCOL]
         for i in range(4)])
    est = _edge_proj(e_attr_pad, epw_q, epb_q)
    return {'xlo': xlo, 'xhi': xhi, 'est': est, 'src3': src3, 'dst3': dst3}


def _layer_weights(layer):
    scale = layer['bn_gamma'] * jax.lax.rsqrt(layer['bn_var'] + BN_EPS)
    shift = layer['bn_beta'] - layer['bn_mean'] * scale
    w1t = layer['w1'].T
    w2t = layer['w2'].T * scale[None, :]
    b2 = (layer['b2'] * scale + shift)[None, :]
    return w1t, layer['b1'][None, :], w2t, b2


def _layer_msg(st):
    xtab = jnp.concatenate(
        [st['xlo'][:, :QCOL], st['xlo'][:, QCOL:],
         st['xhi'][:, :QCOL], st['xhi'][:, QCOL:]], axis=0)
    return _message_pass(xtab, st['est'], st['src3'], st['dst3'])


def kernel(prot_x, prot_edge_index, prot_edge_attr, prot_batch,
           drug_x, drug_edge_index, drug_edge_attr, drug_batch,
           prot_params, drug_params):
    sp = _encoder_prep(prot_params, prot_x, prot_edge_index, prot_edge_attr)
    sd = _encoder_prep(drug_params, drug_x, drug_edge_index, drug_edge_attr)
    for lp, ld in zip(prot_params['layers'], drug_params['layers']):
        ap = _layer_msg(sp)
        ad = _layer_msg(sd)
        sp['xlo'], sp['xhi'] = _mlp(sp['xlo'], sp['xhi'], ap,
                                    *_layer_weights(lp))
        sd['xlo'], sd['xhi'] = _mlp(sd['xlo'], sd['xhi'], ad,
                                    *_layer_weights(ld))
    p = _pool(sp['xlo'], sp['xhi'], prot_batch)
    d = _pool(sd['xlo'], sd['xhi'], drug_batch)
    return (p, d)
